# Initial kernel scaffold; baseline (speedup 1.0000x reference)
import jax
import jax.numpy as jnp
from jax.experimental import pallas as pl


def kernel(features, arg_candidates, predicate_candidates, width_emb, attn_w, attn_b, Wa1, ba1, Wa2, ba2, Wp1, bp1, Wp2, bp2, Ws1, bs1, Ws2, bs2):
    raise NotImplementedError("write your pallas kernel here")


# speedup vs baseline: 6858.4217x; 6858.4217x over previous
import jax
import jax.numpy as jnp
from jax.experimental import pallas as pl

_MAX_W = 8
_N_ARG = 30
_N_PRED = 15


def _span_score_body(has_attn, NB, T, feat_ref, cand_ref, wemb_ref, attnw_ref,
                     attnb_ref, W1_ref, b1_ref, W2_ref, b2_ref,
                     emb_ref, score_ref):
    feats = feat_ref[0]
    cand = cand_ref[0]
    s = cand[:, 0:1]
    e = cand[:, 1:2]
    width = e - s

    iota = jax.lax.broadcasted_iota(jnp.int32, (NB, T), 1)
    oh_s = (iota == s).astype(jnp.float32)
    oh_e = (iota == e).astype(jnp.float32)
    start_f = jnp.dot(oh_s, feats, preferred_element_type=jnp.float32)
    end_f = jnp.dot(oh_e, feats, preferred_element_type=jnp.float32)

    parts = [start_f, end_f]
    if has_attn:
        NWE = wemb_ref.shape[0]
        iota16 = jax.lax.broadcasted_iota(jnp.int32, (NB, NWE), 1)
        wclip = jnp.clip(width, 0, NWE - 1)
        oh_w = (iota16 == wclip).astype(jnp.float32)
        wspan = jnp.dot(oh_w, wemb_ref[...], preferred_element_type=jnp.float32)

        logits = jnp.dot(feats, attnw_ref[...],
                         preferred_element_type=jnp.float32) + attnb_ref[0, 0]
        cols = []
        for j in range(_MAX_W):
            if j == 0:
                cols.append(logits)
            else:
                cols.append(jnp.concatenate(
                    [logits[j:], jnp.zeros((j, 1), jnp.float32)], axis=0))
        L = jnp.concatenate(cols, axis=1)
        span_logits = jnp.dot(oh_s, L, preferred_element_type=jnp.float32)

        jidx = jax.lax.broadcasted_iota(jnp.int32, (NB, _MAX_W), 1)
        m = jidx <= width
        masked = jnp.where(m, span_logits, -1e30)
        mx = jnp.max(masked, axis=1, keepdims=True)
        ex = jnp.exp(masked - mx)
        w = ex / jnp.sum(ex, axis=1, keepdims=True)
        mf = m.astype(jnp.float32)
        w = w * mf
        w = w / jnp.clip(jnp.sum(w, axis=1, keepdims=True), 1e-13)

        M = jnp.zeros((NB, T), jnp.float32)
        for j in range(_MAX_W):
            M = M + jnp.where(iota == (s + j), w[:, j:j + 1], 0.0)
        att = jnp.dot(M, feats, preferred_element_type=jnp.float32)
        parts = [start_f, end_f, wspan, att]

    emb = jnp.concatenate(parts, axis=1)
    emb_ref[0] = emb
    h = jnp.maximum(
        jnp.dot(emb, W1_ref[...], preferred_element_type=jnp.float32)
        + b1_ref[...], 0.0)
    sc = jnp.dot(h, W2_ref[...], preferred_element_type=jnp.float32) + b2_ref[...]
    score_ref[0] = sc


def _span_score(features, cand, wemb, attnw, attnb, W1, b1, W2, b2, has_attn, NB):
    B, T, H = features.shape
    N = cand.shape[1]
    D = W1.shape[0]
    import functools
    body = functools.partial(_span_score_body, has_attn, NB, T)
    grid = (B, N // NB)
    emb, score = pl.pallas_call(
        body,
        grid=grid,
        in_specs=[
            pl.BlockSpec((1, T, H), lambda b, n: (b, 0, 0)),
            pl.BlockSpec((1, NB, 2), lambda b, n: (b, n, 0)),
            pl.BlockSpec(wemb.shape, lambda b, n: (0, 0)),
            pl.BlockSpec(attnw.shape, lambda b, n: (0, 0)),
            pl.BlockSpec(attnb.shape, lambda b, n: (0, 0)),
            pl.BlockSpec(W1.shape, lambda b, n: (0, 0)),
            pl.BlockSpec(b1.shape, lambda b, n: (0, 0)),
            pl.BlockSpec(W2.shape, lambda b, n: (0, 0)),
            pl.BlockSpec(b2.shape, lambda b, n: (0, 0)),
        ],
        out_specs=[
            pl.BlockSpec((1, NB, D), lambda b, n: (b, n, 0)),
            pl.BlockSpec((1, NB, 1), lambda b, n: (b, n, 0)),
        ],
        out_shape=[
            jax.ShapeDtypeStruct((B, N, D), jnp.float32),
            jax.ShapeDtypeStruct((B, N, 1), jnp.float32),
        ],
    )(features, cand, wemb, attnw, attnb, W1, b1, W2, b2)
    return emb, score


def _pair_body(P, A, tp_ref, ta_ref, tps_ref, tas_ref, W1p_ref, W1a_ref,
               b1_ref, W2_ref, b2_ref, out_ref):
    ppart = jnp.dot(tp_ref[0], W1p_ref[...],
                    preferred_element_type=jnp.float32)
    apart = jnp.dot(ta_ref[0], W1a_ref[...],
                    preferred_element_type=jnp.float32)
    tps = tps_ref[0]
    tas = tas_ref[0]
    for p in range(P):
        h = jnp.maximum(apart + ppart[p:p + 1] + b1_ref[...], 0.0)
        sc = jnp.dot(h, W2_ref[...], preferred_element_type=jnp.float32) \
            + b2_ref[...] + tps[p:p + 1, 0:1] + tas
        full = jnp.concatenate(
            [jnp.zeros((sc.shape[0], 1), jnp.float32), sc], axis=1)
        out_ref[0, p] = full


def _pair_scores(tp_emb, ta_emb, tp_s, ta_s, Ws1, bs1, Ws2, bs2):
    B, P, PD = tp_emb.shape
    A = ta_emb.shape[1]
    H = Ws1.shape[1]
    C = Ws2.shape[1] + 1
    W1p = Ws1[:PD]
    W1a = Ws1[PD:]
    import functools
    body = functools.partial(_pair_body, P, A)
    out = pl.pallas_call(
        body,
        grid=(B,),
        in_specs=[
            pl.BlockSpec((1, P, PD), lambda b: (b, 0, 0)),
            pl.BlockSpec((1, A, ta_emb.shape[2]), lambda b: (b, 0, 0)),
            pl.BlockSpec((1, P, 1), lambda b: (b, 0, 0)),
            pl.BlockSpec((1, A, 1), lambda b: (b, 0, 0)),
            pl.BlockSpec(W1p.shape, lambda b: (0, 0)),
            pl.BlockSpec(W1a.shape, lambda b: (0, 0)),
            pl.BlockSpec(bs1.shape, lambda b: (0, 0)),
            pl.BlockSpec(Ws2.shape, lambda b: (0, 0)),
            pl.BlockSpec(bs2.shape, lambda b: (0, 0)),
        ],
        out_specs=[pl.BlockSpec((1, P, A, C), lambda b: (b, 0, 0, 0))],
        out_shape=[jax.ShapeDtypeStruct((B, P, A, C), jnp.float32)],
    )(tp_emb, ta_emb, tp_s, ta_s, W1p, W1a, bs1, Ws2, bs2)
    return out[0]


def _gather_rows(x, idx):
    B, N = idx.shape
    D = x.shape[-1]
    return jnp.take_along_axis(
        x, jnp.broadcast_to(idx[:, :, None], (B, N, D)), axis=1)


def kernel(features, arg_candidates, predicate_candidates, width_emb, attn_w,
           attn_b, Wa1, ba1, Wa2, ba2, Wp1, bp1, Wp2, bp2, Ws1, bs1, Ws2, bs2):
    f32 = jnp.float32
    features = features.astype(f32)
    B = features.shape[0]

    ba1r = ba1.reshape(1, -1)
    ba2r = ba2.reshape(1, -1)
    bp1r = bp1.reshape(1, -1)
    bp2r = bp2.reshape(1, -1)
    bs1r = bs1.reshape(1, -1)
    bs2r = bs2.reshape(1, -1)
    attnbr = attn_b.reshape(1, 1)

    arg_emb, a_full = _span_score(features, arg_candidates, width_emb, attn_w,
                                  attnbr, Wa1, ba1r, Wa2, ba2r,
                                  has_attn=True, NB=512)
    pred_emb, p_full = _span_score(features, predicate_candidates, width_emb,
                                   attn_w, attnbr, Wp1, bp1r, Wp2, bp2r,
                                   has_attn=False, NB=512)

    arg_mask = (arg_candidates[:, :, 1] > 0).astype(f32)
    pred_mask = (predicate_candidates[:, :, 1] > 0).astype(f32)
    n_arg_keep = jnp.minimum((jnp.sum(arg_mask, axis=-1) * 0.8).astype(jnp.int32),
                             _N_ARG)
    n_pred_keep = jnp.minimum((jnp.sum(pred_mask, axis=-1) * 0.4).astype(jnp.int32),
                              _N_PRED)

    def _top(emb, mask, scores, num_keep, max_keep):
        masked_scores = jnp.where(mask[:, :, None] > 0, scores, -1e20)
        _, top_idx = jax.lax.top_k(masked_scores[:, :, 0], max_keep)
        top_idx = jnp.sort(top_idx, axis=1)
        keep_mask = (jnp.arange(max_keep)[None, :] < num_keep[:, None]).astype(f32)
        top_emb = _gather_rows(emb, top_idx)
        top_mask = jnp.take_along_axis(mask, top_idx, axis=1) * keep_mask
        top_scores = jnp.take_along_axis(masked_scores[:, :, 0], top_idx,
                                         axis=1)[:, :, None]
        return top_emb, top_mask, top_idx, top_scores

    ta_emb, ta_mask, ta_idx, ta_scores = _top(arg_emb, arg_mask, a_full,
                                              n_arg_keep, _N_ARG)
    tp_emb, tp_mask, tp_idx, tp_scores = _top(pred_emb, pred_mask, p_full,
                                              n_pred_keep, _N_PRED)

    top_arg_spans = _gather_rows(arg_candidates, ta_idx)
    top_pred_spans = _gather_rows(predicate_candidates, tp_idx)

    srl_scores = _pair_scores(tp_emb, ta_emb, tp_scores, ta_scores,
                              Ws1, bs1r, Ws2, bs2r)

    return (srl_scores, top_pred_spans, top_arg_spans, tp_mask, ta_mask,
            p_full, a_full)
